# baseline (speedup 1.0000x reference)
import functools

import jax
import jax.numpy as jnp
from jax.experimental import pallas as pl
from jax.experimental.pallas import tpu as pltpu

TEMP = 0.1
DT = 0.1
STEPS = 3
EPS = 1e-6

BLOCK_N = 2048
LANES = 128


def _pot_block_kernel(x_ref, c_ref, mus_ref, out_ref):
    x = x_ref[...]
    c = c_ref[...]
    mus = mus_ref[...]
    x2 = jnp.sum(x * x, axis=1, keepdims=True)
    c2 = jnp.sum(c * c, axis=1, keepdims=True)
    xc = jax.lax.dot_general(
        x, c, (((1,), (1,)), ((), ())),
        preferred_element_type=jnp.float32,
        precision=jax.lax.Precision.HIGHEST)
    d2 = x2 + c2.T - 2.0 * xc
    dist = jnp.sqrt(jnp.maximum(d2, 0.0))
    r = 1.0 / (dist + EPS)
    pot = jax.lax.dot_general(
        r, mus, (((1,), (0,)), ((), ())),
        preferred_element_type=jnp.float32,
        precision=jax.lax.Precision.HIGHEST)
    out_ref[...] = pot.reshape(out_ref.shape)


def _query_softmax_kernel(q_ref, c_ref, mus_ref, pot_ref, qout_ref, att_ref):
    z0 = q_ref[...]
    c = c_ref[...]
    mus = mus_ref[...]

    z = z0
    for _ in range(STEPS):
        diff = c - z
        d2 = jnp.sum(diff * diff, axis=1, keepdims=True)
        d = jnp.sqrt(d2)
        w = mus / (d2 * d + EPS)
        flow = jnp.sum(w * diff, axis=0, keepdims=True)
        z = z + DT * flow
    qout_ref[...] = z

    diff0 = c - z0
    d0 = jnp.sqrt(jnp.sum(diff0 * diff0, axis=1, keepdims=True))
    qp = jnp.sum(mus / (d0 + EPS))

    logits = -jnp.abs(qp - pot_ref[...]) / TEMP
    m = jnp.max(logits)
    e = jnp.exp(logits - m)
    att_ref[...] = e / jnp.sum(e)


@functools.partial(jax.jit, static_argnames=())
def kernel(query_z, candidate_z, centers, mus):
    n, d = candidate_z.shape
    k = centers.shape[0]
    mus_col = mus.reshape(k, 1)
    num_blocks = n // BLOCK_N
    rows = BLOCK_N // LANES

    pot = pl.pallas_call(
        _pot_block_kernel,
        grid=(num_blocks,),
        in_specs=[
            pl.BlockSpec((BLOCK_N, d), lambda i: (i, 0)),
            pl.BlockSpec((k, d), lambda i: (0, 0)),
            pl.BlockSpec((k, 1), lambda i: (0, 0)),
        ],
        out_specs=pl.BlockSpec((rows, LANES), lambda i: (i, 0)),
        out_shape=jax.ShapeDtypeStruct((n // LANES, LANES), jnp.float32),
    )(candidate_z, centers, mus_col)

    qout, att = pl.pallas_call(
        _query_softmax_kernel,
        in_specs=[
            pl.BlockSpec((1, d), lambda: (0, 0)),
            pl.BlockSpec((k, d), lambda: (0, 0)),
            pl.BlockSpec((k, 1), lambda: (0, 0)),
            pl.BlockSpec((n // LANES, LANES), lambda: (0, 0)),
        ],
        out_specs=[
            pl.BlockSpec((1, d), lambda: (0, 0)),
            pl.BlockSpec((n // LANES, LANES), lambda: (0, 0)),
        ],
        out_shape=[
            jax.ShapeDtypeStruct((1, d), jnp.float32),
            jax.ShapeDtypeStruct((n // LANES, LANES), jnp.float32),
        ],
    )(query_z, centers, mus_col, pot)

    return qout, att.reshape(n)


# speedup vs baseline: 2.3603x; 2.3603x over previous
import functools

import jax
import jax.numpy as jnp
from jax.experimental import pallas as pl
from jax.experimental.pallas import tpu as pltpu

TEMP = 0.1
DT = 0.1
STEPS = 3
EPS = 1e-6

BLOCK_N = 2048
LANES = 128


def _pot_block_kernel(x_ref, c_ref, mus_ref, out_ref):
    x = x_ref[...]
    c = c_ref[...]
    mus = mus_ref[...]
    x2 = jnp.sum(x * x, axis=1, keepdims=True)
    c2 = jnp.sum(c * c, axis=1, keepdims=True)
    xh = x.astype(jnp.bfloat16)
    xl = (x - xh.astype(jnp.float32)).astype(jnp.bfloat16)
    ch = c.astype(jnp.bfloat16)
    cl = (c - ch.astype(jnp.float32)).astype(jnp.bfloat16)
    dims = (((1,), (1,)), ((), ()))
    xc = jax.lax.dot_general(
        xh, ch, dims, preferred_element_type=jnp.float32)
    xc += jax.lax.dot_general(
        xh, cl, dims, preferred_element_type=jnp.float32)
    xc += jax.lax.dot_general(
        xl, ch, dims, preferred_element_type=jnp.float32)
    d2 = x2 + c2.T - 2.0 * xc
    dist = jnp.sqrt(jnp.maximum(d2, 0.0))
    r = mus.T / (dist + EPS)
    pot = jnp.sum(r, axis=1, keepdims=True)
    out_ref[...] = pot.reshape(out_ref.shape)


def _query_softmax_kernel(q_ref, c_ref, mus_ref, pot_ref, qout_ref, att_ref):
    z0 = q_ref[...]
    c = c_ref[...]
    mus = mus_ref[...]

    z = z0
    for _ in range(STEPS):
        diff = c - z
        d2 = jnp.sum(diff * diff, axis=1, keepdims=True)
        d = jnp.sqrt(d2)
        w = mus / (d2 * d + EPS)
        flow = jnp.sum(w * diff, axis=0, keepdims=True)
        z = z + DT * flow
    qout_ref[...] = z

    diff0 = c - z0
    d0 = jnp.sqrt(jnp.sum(diff0 * diff0, axis=1, keepdims=True))
    qp = jnp.sum(mus / (d0 + EPS))

    logits = -jnp.abs(qp - pot_ref[...]) / TEMP
    m = jnp.max(logits)
    e = jnp.exp(logits - m)
    att_ref[...] = e / jnp.sum(e)


@functools.partial(jax.jit, static_argnames=())
def kernel(query_z, candidate_z, centers, mus):
    n, d = candidate_z.shape
    k = centers.shape[0]
    mus_col = mus.reshape(k, 1)
    num_blocks = n // BLOCK_N
    rows = BLOCK_N // LANES

    pot = pl.pallas_call(
        _pot_block_kernel,
        grid=(num_blocks,),
        in_specs=[
            pl.BlockSpec((BLOCK_N, d), lambda i: (i, 0)),
            pl.BlockSpec((k, d), lambda i: (0, 0)),
            pl.BlockSpec((k, 1), lambda i: (0, 0)),
        ],
        out_specs=pl.BlockSpec((rows, LANES), lambda i: (i, 0)),
        out_shape=jax.ShapeDtypeStruct((n // LANES, LANES), jnp.float32),
    )(candidate_z, centers, mus_col)

    qout, att = pl.pallas_call(
        _query_softmax_kernel,
        in_specs=[
            pl.BlockSpec((1, d), lambda: (0, 0)),
            pl.BlockSpec((k, d), lambda: (0, 0)),
            pl.BlockSpec((k, 1), lambda: (0, 0)),
            pl.BlockSpec((n // LANES, LANES), lambda: (0, 0)),
        ],
        out_specs=[
            pl.BlockSpec((1, d), lambda: (0, 0)),
            pl.BlockSpec((n // LANES, LANES), lambda: (0, 0)),
        ],
        out_shape=[
            jax.ShapeDtypeStruct((1, d), jnp.float32),
            jax.ShapeDtypeStruct((n // LANES, LANES), jnp.float32),
        ],
    )(query_z, centers, mus_col, pot)

    return qout, att.reshape(n)


# speedup vs baseline: 2.8962x; 1.2270x over previous
import functools

import jax
import jax.numpy as jnp
from jax.experimental import pallas as pl
from jax.experimental.pallas import tpu as pltpu

TEMP = 0.1
DT = 0.1
STEPS = 3
EPS = 1e-6

BLOCK_N = 2048
LANES = 128


KCHUNK = 512


def _pot_block_kernel(x_ref, c_ref, mus_ref, out_ref):
    x = x_ref[...]
    bn = x.shape[0]
    x2 = jnp.sum(x * x, axis=1, keepdims=True)
    xm = -2.0 * x
    xh = xm.astype(jnp.bfloat16)
    xl = (xm - xh.astype(jnp.float32)).astype(jnp.bfloat16)
    dims = (((1,), (1,)), ((), ()))
    k = c_ref.shape[0]
    pot = jnp.zeros((bn, 1), jnp.float32)
    for kc in range(k // KCHUNK):
        c = c_ref[pl.ds(kc * KCHUNK, KCHUNK), :]
        mus = mus_ref[pl.ds(kc * KCHUNK, KCHUNK), :]
        c2 = jnp.sum(c * c, axis=1, keepdims=True)
        ch = c.astype(jnp.bfloat16)
        cl = (c - ch.astype(jnp.float32)).astype(jnp.bfloat16)
        xc = jax.lax.dot_general(
            xh, ch, dims, preferred_element_type=jnp.float32)
        xc += jax.lax.dot_general(
            xh, cl, dims, preferred_element_type=jnp.float32)
        xc += jax.lax.dot_general(
            xl, ch, dims, preferred_element_type=jnp.float32)
        d2 = (x2 + c2.T) + xc
        r = mus.T * jax.lax.rsqrt(jnp.maximum(d2, 1e-12))
        pot += jnp.sum(r, axis=1, keepdims=True)
    out_ref[...] = pot.reshape(out_ref.shape)


def _query_softmax_kernel(q_ref, c_ref, mus_ref, pot_ref, qout_ref, att_ref):
    z0 = q_ref[...]
    c = c_ref[...]
    mus = mus_ref[...]

    z = z0
    for _ in range(STEPS):
        diff = c - z
        d2 = jnp.sum(diff * diff, axis=1, keepdims=True)
        d = jnp.sqrt(d2)
        w = mus / (d2 * d + EPS)
        flow = jnp.sum(w * diff, axis=0, keepdims=True)
        z = z + DT * flow
    qout_ref[...] = z

    diff0 = c - z0
    d0 = jnp.sqrt(jnp.sum(diff0 * diff0, axis=1, keepdims=True))
    qp = jnp.sum(mus / (d0 + EPS))

    logits = -jnp.abs(qp - pot_ref[...]) / TEMP
    m = jnp.max(logits)
    e = jnp.exp(logits - m)
    att_ref[...] = e / jnp.sum(e)


@functools.partial(jax.jit, static_argnames=())
def kernel(query_z, candidate_z, centers, mus):
    n, d = candidate_z.shape
    k = centers.shape[0]
    mus_col = mus.reshape(k, 1)
    num_blocks = n // BLOCK_N
    rows = BLOCK_N // LANES

    pot = pl.pallas_call(
        _pot_block_kernel,
        grid=(num_blocks,),
        in_specs=[
            pl.BlockSpec((BLOCK_N, d), lambda i: (i, 0)),
            pl.BlockSpec((k, d), lambda i: (0, 0)),
            pl.BlockSpec((k, 1), lambda i: (0, 0)),
        ],
        out_specs=pl.BlockSpec((rows, LANES), lambda i: (i, 0)),
        out_shape=jax.ShapeDtypeStruct((n // LANES, LANES), jnp.float32),
    )(candidate_z, centers, mus_col)

    qout, att = pl.pallas_call(
        _query_softmax_kernel,
        in_specs=[
            pl.BlockSpec((1, d), lambda: (0, 0)),
            pl.BlockSpec((k, d), lambda: (0, 0)),
            pl.BlockSpec((k, 1), lambda: (0, 0)),
            pl.BlockSpec((n // LANES, LANES), lambda: (0, 0)),
        ],
        out_specs=[
            pl.BlockSpec((1, d), lambda: (0, 0)),
            pl.BlockSpec((n // LANES, LANES), lambda: (0, 0)),
        ],
        out_shape=[
            jax.ShapeDtypeStruct((1, d), jnp.float32),
            jax.ShapeDtypeStruct((n // LANES, LANES), jnp.float32),
        ],
    )(query_z, centers, mus_col, pot)

    return qout, att.reshape(n)
